# Initial kernel scaffold; baseline (speedup 1.0000x reference)
import jax
import jax.numpy as jnp
from jax.experimental import pallas as pl


def kernel(x, W_gender, W_age, W_occupation, W_area):
    raise NotImplementedError("write your pallas kernel here")


# speedup vs baseline: 1.2280x; 1.2280x over previous
import functools

import jax
import jax.numpy as jnp
from jax import lax
from jax.experimental import pallas as pl
from jax.experimental.pallas import tpu as pltpu
from jax.experimental.pallas import tpu_sc as plsc

B = 16384
D = 64
NT = 3

_info = plsc.get_sparse_core_info()
NC, NS = _info.num_cores, _info.num_subcores
NW = NC * NS
BPW = B // NW
CHUNK = 128
NCHUNK = BPW // CHUNK

_mesh = plsc.VectorSubcoreMesh(core_axis_name="c", subcore_axis_name="s")


@functools.partial(
    pl.kernel,
    mesh=_mesh,
    out_type=tuple(
        jax.ShapeDtypeStruct((B, 2 * D), jnp.float32) for _ in range(NT)
    ),
    scratch_types=[
        pltpu.VMEM((NT, CHUNK), jnp.int32),
        pltpu.VMEM((NT, CHUNK, 2 * D), jnp.float32),
        pltpu.SemaphoreType.DMA,
    ],
)
def _emb_gather(xI_hbm, wga_hbm, wo2_hbm, wzp_hbm,
                oga_hbm, oo_hbm, oz_hbm,
                idx_v, rows_v, sem):
    wid = lax.axis_index("s") * NC + lax.axis_index("c")
    base = wid * BPW
    tables = (wga_hbm, wo2_hbm, wzp_hbm)
    outs = (oga_hbm, oo_hbm, oz_hbm)
    for c in range(NCHUNK):
        row0 = base + c * CHUNK
        pltpu.sync_copy(xI_hbm.at[:, pl.ds(row0, CHUNK)], idx_v)
        copies = [
            pltpu.async_copy(tables[k].at[idx_v.at[k]], rows_v.at[k], sem)
            for k in range(NT)
        ]
        for cp in copies:
            cp.wait()
        for k in range(NT):
            pltpu.sync_copy(rows_v.at[k], outs[k].at[pl.ds(row0, CHUNK)])


def kernel(x, W_gender, W_age, W_occupation, W_area):
    xi = x.astype(jnp.int32)
    idx_ga = xi[:, 0] * 7 + xi[:, 1]
    idx_o = xi[:, 2]
    idx_zp = xi[:, 3] >> 1
    parity = xi[:, 3] & 1
    xI = jnp.stack((idx_ga, idx_o, idx_zp))
    w_ga = jnp.concatenate(
        (jnp.repeat(W_gender, 7, axis=0), jnp.tile(W_age, (2, 1))), axis=1
    )
    w_o2 = jnp.concatenate((W_occupation, W_occupation), axis=1)
    w_zp = jnp.reshape(W_area, (W_area.shape[0] // 2, 2 * D))
    ga, o2, zp = _emb_gather(xI, w_ga, w_o2, w_zp)
    z = jnp.where((parity == 1)[:, None], zp[:, D:], zp[:, :D])
    return jnp.concatenate((ga, o2[:, :D], z), axis=1)


# speedup vs baseline: 1.2802x; 1.0425x over previous
import functools

import jax
import jax.numpy as jnp
from jax import lax
from jax.experimental import pallas as pl
from jax.experimental.pallas import tpu as pltpu
from jax.experimental.pallas import tpu_sc as plsc

B = 16384
D = 64
NT = 3

_info = plsc.get_sparse_core_info()
NC, NS = _info.num_cores, _info.num_subcores
NW = NC * NS
BPW = B // NW
CHUNK = 128
NCHUNK = BPW // CHUNK

_mesh = plsc.VectorSubcoreMesh(core_axis_name="c", subcore_axis_name="s")


@functools.partial(
    pl.kernel,
    mesh=_mesh,
    out_type=tuple(
        jax.ShapeDtypeStruct((B, 2 * D), jnp.float32) for _ in range(NT)
    ),
    scratch_types=[
        pltpu.VMEM((2, NT, CHUNK), jnp.int32),
        pltpu.VMEM((2, NT, CHUNK, 2 * D), jnp.float32),
        pltpu.SemaphoreType.DMA((2,)),
        pltpu.SemaphoreType.DMA((2,)),
        pltpu.SemaphoreType.DMA((2,)),
    ],
)
def _emb_gather(iga_hbm, io_hbm, izp_hbm, wga_hbm, wo2_hbm, wzp_hbm,
                oga_hbm, oo_hbm, oz_hbm,
                idx_v, rows_v, si, sg, so):
    wid = lax.axis_index("s") * NC + lax.axis_index("c")
    base = wid * BPW
    idxs = (iga_hbm, io_hbm, izp_hbm)
    tables = (wga_hbm, wo2_hbm, wzp_hbm)
    outs = (oga_hbm, oo_hbm, oz_hbm)

    def idx_copies(c):
        p = c % 2
        return [
            pltpu.async_copy(
                idxs[k].at[pl.ds(base + c * CHUNK, CHUNK)],
                idx_v.at[p, k], si.at[p])
            for k in range(NT)
        ]

    def gather_copies(c):
        p = c % 2
        return [
            pltpu.async_copy(
                tables[k].at[idx_v.at[p, k]], rows_v.at[p, k], sg.at[p])
            for k in range(NT)
        ]

    def out_copies(c):
        p = c % 2
        return [
            pltpu.async_copy(
                rows_v.at[p, k], outs[k].at[pl.ds(base + c * CHUNK, CHUNK)],
                so.at[p])
            for k in range(NT)
        ]

    ic = {0: idx_copies(0)}
    for cp in ic[0]:
        cp.wait()
    g = {0: gather_copies(0)}
    if NCHUNK > 1:
        ic[1] = idx_copies(1)
    o = {}
    for c in range(NCHUNK):
        if c >= 1:
            for cp in o[c - 1]:
                cp.wait()
        if c + 1 < NCHUNK:
            for cp in ic[c + 1]:
                cp.wait()
            g[c + 1] = gather_copies(c + 1)
        for cp in g[c]:
            cp.wait()
        if c + 2 < NCHUNK:
            ic[c + 2] = idx_copies(c + 2)
        o[c] = out_copies(c)
    for cp in o[NCHUNK - 1]:
        cp.wait()


def kernel(x, W_gender, W_age, W_occupation, W_area):
    xi = x.astype(jnp.int32)
    idx_ga = xi[:, 0] * 7 + xi[:, 1]
    idx_o = xi[:, 2]
    idx_zp = xi[:, 3] >> 1
    parity = xi[:, 3] & 1
    w_ga = jnp.concatenate(
        (jnp.repeat(W_gender, 7, axis=0), jnp.tile(W_age, (2, 1))), axis=1
    )
    w_o2 = jnp.concatenate((W_occupation, W_occupation), axis=1)
    w_zp = jnp.reshape(W_area, (W_area.shape[0] // 2, 2 * D))
    ga, o2, zp = _emb_gather(idx_ga, idx_o, idx_zp, w_ga, w_o2, w_zp)
    z = jnp.where((parity == 1)[:, None], zp[:, D:], zp[:, :D])
    return jnp.concatenate((ga, o2[:, :D], z), axis=1)


# speedup vs baseline: 1.8487x; 1.4441x over previous
import functools

import jax
import jax.numpy as jnp
from jax import lax
from jax.experimental import pallas as pl
from jax.experimental.pallas import tpu as pltpu
from jax.experimental.pallas import tpu_sc as plsc

B = 16384
D = 64
NT = 3

_info = plsc.get_sparse_core_info()
NC, NS = _info.num_cores, _info.num_subcores
NW = NC * NS
BPW = B // NW
CHUNK = 128
NCHUNK = BPW // CHUNK

_mesh = plsc.VectorSubcoreMesh(core_axis_name="c", subcore_axis_name="s")


@functools.partial(
    pl.kernel,
    mesh=_mesh,
    out_type=tuple(
        jax.ShapeDtypeStruct((B, 2 * D), jnp.float32) for _ in range(NT)
    ),
    scratch_types=[
        pltpu.VMEM((2, NT, CHUNK), jnp.int32),
        pltpu.VMEM((2, NT, CHUNK, 2 * D), jnp.float32),
        pltpu.SemaphoreType.DMA((2,)),
        pltpu.SemaphoreType.DMA((2,)),
        pltpu.SemaphoreType.DMA((2,)),
    ],
)
def _emb_gather(iga_hbm, io_hbm, izp_hbm, wga_hbm, wo2_hbm, wzp_hbm,
                oga_hbm, oo_hbm, oz_hbm,
                idx_v, rows_v, si, sg, so):
    wid = lax.axis_index("s") * NC + lax.axis_index("c")
    base = wid * BPW
    idxs = (iga_hbm, io_hbm, izp_hbm)
    tables = (wga_hbm, wo2_hbm, wzp_hbm)
    outs = (oga_hbm, oo_hbm, oz_hbm)

    def idx_copies(c):
        p = c % 2
        return [
            pltpu.async_copy(
                idxs[k].at[pl.ds(base + c * CHUNK, CHUNK)],
                idx_v.at[p, k], si.at[p])
            for k in range(NT)
        ]

    def gather_copies(c):
        p = c % 2
        return [
            pltpu.async_copy(
                tables[k].at[idx_v.at[p, k]], rows_v.at[p, k], sg.at[p])
            for k in range(NT)
        ]

    def out_copies(c):
        p = c % 2
        return [
            pltpu.async_copy(
                rows_v.at[p, k], outs[k].at[pl.ds(base + c * CHUNK, CHUNK)],
                so.at[p])
            for k in range(NT)
        ]

    ic = {0: idx_copies(0)}
    for cp in ic[0]:
        cp.wait()
    g = {0: gather_copies(0)}
    if NCHUNK > 1:
        ic[1] = idx_copies(1)
    o = {}
    for c in range(NCHUNK):
        if c >= 1:
            for cp in o[c - 1]:
                cp.wait()
        if c + 1 < NCHUNK:
            for cp in ic[c + 1]:
                cp.wait()
            g[c + 1] = gather_copies(c + 1)
        for cp in g[c]:
            cp.wait()
        if c + 2 < NCHUNK:
            ic[c + 2] = idx_copies(c + 2)
        o[c] = out_copies(c)
    for cp in o[NCHUNK - 1]:
        cp.wait()


def kernel(x, W_gender, W_age, W_occupation, W_area):
    xi = x.astype(jnp.int32)
    wid_of_row = jnp.arange(B, dtype=jnp.int32) // BPW
    idx_ga = xi[:, 0] * 7 + xi[:, 1] + wid_of_row * 14
    idx_o = xi[:, 2] + wid_of_row * 21
    idx_zp = xi[:, 3] >> 1
    parity = xi[:, 3] & 1
    w_ga = jnp.tile(jnp.concatenate(
        (jnp.repeat(W_gender, 7, axis=0), jnp.tile(W_age, (2, 1))), axis=1
    ), (NW, 1))
    w_o2 = jnp.tile(
        jnp.concatenate((W_occupation, W_occupation), axis=1), (NW, 1)
    )
    w_zp = jnp.reshape(W_area, (W_area.shape[0] // 2, 2 * D))
    ga, o2, zp = _emb_gather(idx_ga, idx_o, idx_zp, w_ga, w_o2, w_zp)
    z = jnp.where((parity == 1)[:, None], zp[:, D:], zp[:, :D])
    return jnp.concatenate((ga, o2[:, :D], z), axis=1)


# speedup vs baseline: 2.4195x; 1.3088x over previous
import functools

import jax
import jax.numpy as jnp
from jax import lax
from jax.experimental import pallas as pl
from jax.experimental.pallas import tpu as pltpu
from jax.experimental.pallas import tpu_sc as plsc

B = 16384
D = 64

_info = plsc.get_sparse_core_info()
NC, NS = _info.num_cores, _info.num_subcores
NW = NC * NS
BPW = B // NW
CHUNK = 128
NCHUNK = BPW // CHUNK

_mesh = plsc.VectorSubcoreMesh(core_axis_name="c", subcore_axis_name="s")


@functools.partial(
    pl.kernel,
    mesh=_mesh,
    out_type=jax.ShapeDtypeStruct((B, 4 * D), jnp.float32),
    scratch_types=[
        pltpu.VMEM((2, 3, CHUNK), jnp.int32),
        pltpu.VMEM((2, CHUNK, 4 * D), jnp.float32),
        pltpu.VMEM((2, CHUNK, 2 * D), jnp.float32),
        pltpu.SemaphoreType.DMA((2,)),
        pltpu.SemaphoreType.DMA((2,)),
        pltpu.SemaphoreType.DMA((2,)),
    ],
)
def _emb_fused(iga_hbm, io_hbm, iz_hbm, wga_hbm, wo2_hbm, wz2_hbm,
               out_hbm, idx_v, stg_v, zb_v, si, sg, so):
    wid = lax.axis_index("s") * NC + lax.axis_index("c")

    def idx_copies(c):
        p = c % 2
        gc = wid * NCHUNK + c
        return [
            pltpu.async_copy(iga_hbm.at[gc], idx_v.at[p, 0], si.at[p]),
            pltpu.async_copy(io_hbm.at[gc], idx_v.at[p, 1], si.at[p]),
            pltpu.async_copy(iz_hbm.at[gc], idx_v.at[p, 2], si.at[p]),
        ]

    def gather_copies(c):
        p = c % 2
        stg = stg_v.at[p]
        return [
            pltpu.async_copy(wga_hbm.at[idx_v.at[p, 0]],
                             stg.at[:, pl.ds(0, 128)], sg.at[p]),
            pltpu.async_copy(wo2_hbm.at[idx_v.at[p, 1]],
                             stg.at[:, pl.ds(128, 128)], sg.at[p]),
            pltpu.async_copy(wz2_hbm.at[idx_v.at[p, 2]], zb_v.at[p],
                             sg.at[p]),
        ]

    def assemble(c):
        p = c % 2

        def body(r, _):
            for j in range(4):
                stg_v[p, r, pl.ds(3 * D + 16 * j, 16)] = (
                    zb_v[p, r, pl.ds(16 * j, 16)])
            return 0

        lax.fori_loop(0, CHUNK, body, 0, unroll=2)

    def out_copy(c):
        p = c % 2
        return pltpu.async_copy(
            stg_v.at[p],
            out_hbm.at[pl.ds((wid * NCHUNK + c) * CHUNK, CHUNK)], so.at[p])

    ic = {0: idx_copies(0)}
    for cp in ic[0]:
        cp.wait()
    g = {0: gather_copies(0)}
    if NCHUNK > 1:
        ic[1] = idx_copies(1)
    o = {}
    for c in range(NCHUNK):
        if c >= 1:
            for cp in o[c - 1]:
                cp.wait()
        if c + 1 < NCHUNK:
            for cp in ic[c + 1]:
                cp.wait()
            g[c + 1] = gather_copies(c + 1)
        for cp in g[c]:
            cp.wait()
        if c + 2 < NCHUNK:
            ic[c + 2] = idx_copies(c + 2)
        assemble(c)
        o[c] = [out_copy(c)]
    for cp in o[NCHUNK - 1]:
        cp.wait()


def kernel(x, W_gender, W_age, W_occupation, W_area):
    xi = x.astype(jnp.int32)
    wid_of_row = jnp.arange(B, dtype=jnp.int32) // BPW
    idx_ga = (xi[:, 0] * 7 + xi[:, 1] + wid_of_row * 14).reshape(B // 128, 128)
    idx_o = (xi[:, 2] + wid_of_row * 21).reshape(B // 128, 128)
    idx_z = xi[:, 3].reshape(B // 128, 128)
    w_ga = jnp.tile(jnp.concatenate(
        (jnp.repeat(W_gender, 7, axis=0), jnp.tile(W_age, (2, 1))), axis=1
    ), (NW, 1))
    w_o2 = jnp.tile(
        jnp.concatenate((W_occupation, W_occupation), axis=1), (NW, 1)
    )
    w_z2 = jnp.pad(W_area, ((0, 0), (0, D)))
    return _emb_fused(idx_ga, idx_o, idx_z, w_ga, w_o2, w_z2)
